# baseline (speedup 1.0000x reference)
import jax
import jax.numpy as jnp
from jax.experimental import pallas as pl
from jax.experimental.pallas import tpu as pltpu

_CHUNK = 3200000
_NSTEPS = 2


def _body(k_ref, b_hbm, o_ref, scratch, sems):
    i = pl.program_id(0)
    k = k_ref[0]

    @pl.when(i == 0)
    def _enqueue_all():
        for j in range(_NSTEPS):
            pltpu.make_async_copy(
                b_hbm.at[k, pl.ds(j * _CHUNK, _CHUNK)],
                scratch.at[pl.ds(j * _CHUNK, _CHUNK)], sems.at[j],
            ).start()

    pltpu.make_async_copy(
        b_hbm.at[k, pl.ds(i * _CHUNK, _CHUNK)],
        scratch.at[pl.ds(i * _CHUNK, _CHUNK)], sems.at[i],
    ).wait()
    o_ref[...] = jax.nn.sigmoid(scratch[pl.ds(i * _CHUNK, _CHUNK)])


def kernel(B, k, edge_index, n):
    E = B.shape[1]
    k_arr = jnp.atleast_1d(k).astype(jnp.int32)
    out = pl.pallas_call(
        _body,
        grid_spec=pltpu.PrefetchScalarGridSpec(
            num_scalar_prefetch=1,
            grid=(_NSTEPS,),
            in_specs=[pl.BlockSpec(memory_space=pl.ANY)],
            out_specs=pl.BlockSpec((_CHUNK,), lambda i, kref: (i,)),
            scratch_shapes=[
                pltpu.VMEM((_NSTEPS * _CHUNK,), jnp.float32),
                pltpu.SemaphoreType.DMA((_NSTEPS,)),
            ],
        ),
        out_shape=jax.ShapeDtypeStruct((E,), jnp.float32),
    )(k_arr, B)
    return out


# speedup vs baseline: 1.0986x; 1.0986x over previous
import jax
import jax.numpy as jnp
from jax.experimental import pallas as pl
from jax.experimental.pallas import tpu as pltpu

_E = 6400000
_BLK = 640000
_NBLK = _E // _BLK
_DMA_BLKS = (3, 3, 2, 1, 1)
_DMA_STARTS = (0, 3, 6, 8, 9)
_NDMA = len(_DMA_BLKS)


def _body(k_ref, b_hbm, o_hbm, scratch, out_vmem, in_sems, out_sems):
    k = k_ref[0]
    for d in range(_NDMA):
        lo = _DMA_STARTS[d] * _BLK
        sz = _DMA_BLKS[d] * _BLK
        pltpu.make_async_copy(
            b_hbm.at[k, pl.ds(lo, sz)],
            scratch.at[pl.ds(lo, sz)], in_sems.at[d],
        ).start()

    for blk in range(_NBLK):
        if blk in _DMA_STARTS:
            d = _DMA_STARTS.index(blk)
            lo = _DMA_STARTS[d] * _BLK
            sz = _DMA_BLKS[d] * _BLK
            pltpu.make_async_copy(
                b_hbm.at[k, pl.ds(lo, sz)],
                scratch.at[pl.ds(lo, sz)], in_sems.at[d],
            ).wait()
        off = blk * _BLK
        out_vmem[pl.ds(off, _BLK)] = jax.nn.sigmoid(scratch[pl.ds(off, _BLK)])
        pltpu.make_async_copy(
            out_vmem.at[pl.ds(off, _BLK)],
            o_hbm.at[pl.ds(off, _BLK)], out_sems.at[blk],
        ).start()

    for blk in range(_NBLK):
        off = blk * _BLK
        pltpu.make_async_copy(
            out_vmem.at[pl.ds(off, _BLK)],
            o_hbm.at[pl.ds(off, _BLK)], out_sems.at[blk],
        ).wait()


def kernel(B, k, edge_index, n):
    k_arr = jnp.atleast_1d(k).astype(jnp.int32)
    out = pl.pallas_call(
        _body,
        grid_spec=pltpu.PrefetchScalarGridSpec(
            num_scalar_prefetch=1,
            grid=(1,),
            in_specs=[pl.BlockSpec(memory_space=pl.ANY)],
            out_specs=pl.BlockSpec(memory_space=pl.ANY),
            scratch_shapes=[
                pltpu.VMEM((_E,), jnp.float32),
                pltpu.VMEM((_E,), jnp.float32),
                pltpu.SemaphoreType.DMA((_NDMA,)),
                pltpu.SemaphoreType.DMA((_NBLK,)),
            ],
        ),
        out_shape=jax.ShapeDtypeStruct((_E,), jnp.float32),
    )(k_arr, B)
    return out
